# baseline (speedup 1.0000x reference)
import jax
import jax.numpy as jnp
from jax.experimental import pallas as pl
from jax.experimental.pallas import tpu as pltpu

ALPHA = 0.25
LOSS_CONF_W = 1.0 * 1.5
LOSS_CLS_W = 1.0
LOSS_REG_W = 5.0 * 1.2
N = 134400
G = 5
BN = N // G
CH = 128
NC = BN // CH


def _focal(x, t):
    q = 0.5 * jnp.tanh(0.5 * x)
    u = 1.0 - (t + t)
    nce = jnp.log(0.5 - q) + x * t
    one_m_pt = 0.5 + q * u
    neg_alpha_t = -0.5 - 0.25 * u
    return neg_alpha_t * nce * one_m_pt * one_m_pt


def _loss_kernel(conf_ref, clsp_ref, clst_ref, boxp_ref, boxt_ref,
                 fg_ref, aw_ref,
                 oconf_ref, ocls_ref, obox_ref, otot_ref, acc_ref, acc8_ref):
    i = pl.program_id(0)

    @pl.when(i == 0)
    def _init():
        acc_ref[...] = jnp.zeros_like(acc_ref)
        acc8_ref[...] = jnp.zeros_like(acc8_ref)

    a_conf = acc_ref[0:1, :]
    a_cls = acc8_ref[...]
    a_box = acc_ref[2:3, :]
    a_fg = acc_ref[3:4, :]
    eps = 1e-7
    fgrow = fg_ref[0]
    for k in range(NC):
        s = slice(k * CH, (k + 1) * CH)
        fg = fgrow[:, s]

        a_conf = a_conf + _focal(conf_ref[0:1, s], fg)

        f = _focal(clsp_ref[:, s], clst_ref[:, s])
        cs = (((f[0:8] + f[8:16]) + (f[16:24] + f[24:32]))
              + ((f[32:40] + f[40:48]) + (f[48:56] + f[56:64]))
              + (f[64:72] + f[72:80]))
        a_cls = a_cls + fg * cs

        px1, py1 = boxp_ref[0:1, s], boxp_ref[1:2, s]
        px2, py2 = boxp_ref[2:3, s], boxp_ref[3:4, s]
        tx1, ty1 = boxt_ref[0:1, s], boxt_ref[1:2, s]
        tx2, ty2 = boxt_ref[2:3, s], boxt_ref[3:4, s]
        area_p = jnp.maximum(px2 - px1, 0.0) * jnp.maximum(py2 - py1, 0.0)
        area_t = jnp.maximum(tx2 - tx1, 0.0) * jnp.maximum(ty2 - ty1, 0.0)
        inter = (jnp.maximum(jnp.minimum(px2, tx2) - jnp.maximum(px1, tx1),
                             0.0)
                 * jnp.maximum(jnp.minimum(py2, ty2) - jnp.maximum(py1, ty1),
                               0.0))
        union = area_p + area_t - inter + eps
        iou = inter / union
        c_area = ((jnp.maximum(px2, tx2) - jnp.minimum(px1, tx1))
                  * (jnp.maximum(py2, ty2) - jnp.minimum(py1, ty1)) + eps)
        giou = iou - (c_area - union) / c_area
        a_box = a_box + (1.0 - giou) * fg
        a_fg = a_fg + fg
    acc_ref[0:1, :] = a_conf
    acc8_ref[...] = a_cls
    acc_ref[2:3, :] = a_box
    acc_ref[3:4, :] = a_fg

    @pl.when(i == G - 1)
    def _finish():
        sum_conf = jnp.sum(acc_ref[0])
        sum_cls = jnp.sum(acc8_ref[...])
        sum_box = jnp.sum(acc_ref[2])
        num_fg = jnp.maximum(jnp.sum(acc_ref[3]), 1.0)
        lc = sum_conf / num_fg
        lcl = sum_cls / num_fg
        lb = sum_box / num_fg
        aw = aw_ref[...]
        ew = jnp.exp(aw - jnp.max(aw))
        w = ew / jnp.sum(ew)
        lane = jax.lax.broadcasted_iota(jnp.int32, (1, 3), 1)
        w0 = jnp.sum(jnp.where(lane == 0, w, 0.0))
        w1 = jnp.sum(jnp.where(lane == 1, w, 0.0))
        w2 = jnp.sum(jnp.where(lane == 2, w, 0.0))
        tot = (w0 * LOSS_CONF_W * lc + w1 * LOSS_CLS_W * lcl
               + w2 * LOSS_REG_W * lb)
        oconf_ref[...] = jnp.reshape(lc, (1, 1))
        ocls_ref[...] = jnp.reshape(lcl, (1, 1))
        obox_ref[...] = jnp.reshape(lb, (1, 1))
        otot_ref[...] = jnp.reshape(tot, (1, 1))


def kernel(conf_preds, cls_preds, box_preds, cls_targets, box_targets,
           fg_mask, adaptive_weight):
    conf_t = conf_preds.T
    clsp_t = cls_preds.T
    clst_t = cls_targets.T
    boxp_t = box_preds.T
    boxt_t = box_targets.T
    fg3 = fg_mask.astype(jnp.float32).reshape(G, 1, BN)
    aw2 = adaptive_weight.reshape(1, 3)

    out_spec = pl.BlockSpec((1, 1), lambda i: (0, 0))

    outs = pl.pallas_call(
        _loss_kernel,
        grid=(G,),
        in_specs=[
            pl.BlockSpec((1, BN), lambda i: (0, i)),
            pl.BlockSpec((80, BN), lambda i: (0, i)),
            pl.BlockSpec((80, BN), lambda i: (0, i)),
            pl.BlockSpec((4, BN), lambda i: (0, i)),
            pl.BlockSpec((4, BN), lambda i: (0, i)),
            pl.BlockSpec((1, 1, BN), lambda i: (i, 0, 0)),
            pl.BlockSpec((1, 3), lambda i: (0, 0)),
        ],
        out_specs=[out_spec, out_spec, out_spec, out_spec],
        out_shape=[jax.ShapeDtypeStruct((1, 1), jnp.float32)] * 4,
        scratch_shapes=[
            pltpu.VMEM((4, CH), jnp.float32),
            pltpu.VMEM((8, CH), jnp.float32),
        ],
        compiler_params=pltpu.CompilerParams(
            dimension_semantics=("arbitrary",),
        ),
    )(conf_t, clsp_t, clst_t, boxp_t, boxt_t, fg3, aw2)

    oc, ocl, ob, ot = outs
    return (oc.reshape(()), ocl.reshape(()), ob.reshape(()), ot.reshape(()))


# speedup vs baseline: 1.0587x; 1.0587x over previous
import jax
import jax.numpy as jnp
from jax.experimental import pallas as pl
from jax.experimental.pallas import tpu as pltpu

ALPHA = 0.25
LOSS_CONF_W = 1.0 * 1.5
LOSS_CLS_W = 1.0
LOSS_REG_W = 5.0 * 1.2
N = 134400
G = 10
BN = N // G
CH = 128
NC = BN // CH


def _focal(x, t):
    q = 0.5 * jnp.tanh(0.5 * x)
    u = 1.0 - (t + t)
    nce = jnp.log(0.5 - q) + x * t
    one_m_pt = 0.5 + q * u
    neg_alpha_t = -0.5 - 0.25 * u
    return neg_alpha_t * nce * one_m_pt * one_m_pt


def _loss_kernel(conf_ref, clsp_ref, clst_ref, boxp_ref, boxt_ref,
                 fg_ref, aw_ref,
                 oconf_ref, ocls_ref, obox_ref, otot_ref, acc_ref, acc8_ref):
    i = pl.program_id(0)

    @pl.when(i == 0)
    def _init():
        acc_ref[...] = jnp.zeros_like(acc_ref)
        acc8_ref[...] = jnp.zeros_like(acc8_ref)

    a_conf = acc_ref[0:1, :]
    a_cls = acc8_ref[...]
    a_box = acc_ref[2:3, :]
    a_fg = acc_ref[3:4, :]
    eps = 1e-7
    fgrow = fg_ref[0]
    for k in range(NC):
        s = slice(k * CH, (k + 1) * CH)
        fg = fgrow[:, s]

        a_conf = a_conf + _focal(conf_ref[0:1, s], fg)

        f = _focal(clsp_ref[:, s], clst_ref[:, s])
        cs = (((f[0:8] + f[8:16]) + (f[16:24] + f[24:32]))
              + ((f[32:40] + f[40:48]) + (f[48:56] + f[56:64]))
              + (f[64:72] + f[72:80]))
        a_cls = a_cls + fg * cs

        px1, py1 = boxp_ref[0:1, s], boxp_ref[1:2, s]
        px2, py2 = boxp_ref[2:3, s], boxp_ref[3:4, s]
        tx1, ty1 = boxt_ref[0:1, s], boxt_ref[1:2, s]
        tx2, ty2 = boxt_ref[2:3, s], boxt_ref[3:4, s]
        area_p = jnp.maximum(px2 - px1, 0.0) * jnp.maximum(py2 - py1, 0.0)
        area_t = jnp.maximum(tx2 - tx1, 0.0) * jnp.maximum(ty2 - ty1, 0.0)
        inter = (jnp.maximum(jnp.minimum(px2, tx2) - jnp.maximum(px1, tx1),
                             0.0)
                 * jnp.maximum(jnp.minimum(py2, ty2) - jnp.maximum(py1, ty1),
                               0.0))
        union = area_p + area_t - inter + eps
        iou = inter / union
        c_area = ((jnp.maximum(px2, tx2) - jnp.minimum(px1, tx1))
                  * (jnp.maximum(py2, ty2) - jnp.minimum(py1, ty1)) + eps)
        giou = iou - (c_area - union) / c_area
        a_box = a_box + (1.0 - giou) * fg
        a_fg = a_fg + fg
    acc_ref[0:1, :] = a_conf
    acc8_ref[...] = a_cls
    acc_ref[2:3, :] = a_box
    acc_ref[3:4, :] = a_fg

    @pl.when(i == G - 1)
    def _finish():
        sum_conf = jnp.sum(acc_ref[0])
        sum_cls = jnp.sum(acc8_ref[...])
        sum_box = jnp.sum(acc_ref[2])
        num_fg = jnp.maximum(jnp.sum(acc_ref[3]), 1.0)
        lc = sum_conf / num_fg
        lcl = sum_cls / num_fg
        lb = sum_box / num_fg
        aw = aw_ref[...]
        ew = jnp.exp(aw - jnp.max(aw))
        w = ew / jnp.sum(ew)
        lane = jax.lax.broadcasted_iota(jnp.int32, (1, 3), 1)
        w0 = jnp.sum(jnp.where(lane == 0, w, 0.0))
        w1 = jnp.sum(jnp.where(lane == 1, w, 0.0))
        w2 = jnp.sum(jnp.where(lane == 2, w, 0.0))
        tot = (w0 * LOSS_CONF_W * lc + w1 * LOSS_CLS_W * lcl
               + w2 * LOSS_REG_W * lb)
        oconf_ref[...] = jnp.reshape(lc, (1, 1))
        ocls_ref[...] = jnp.reshape(lcl, (1, 1))
        obox_ref[...] = jnp.reshape(lb, (1, 1))
        otot_ref[...] = jnp.reshape(tot, (1, 1))


def kernel(conf_preds, cls_preds, box_preds, cls_targets, box_targets,
           fg_mask, adaptive_weight):
    conf_t = conf_preds.T
    clsp_t = cls_preds.T
    clst_t = cls_targets.T
    boxp_t = box_preds.T
    boxt_t = box_targets.T
    fg3 = fg_mask.astype(jnp.float32).reshape(G, 1, BN)
    aw2 = adaptive_weight.reshape(1, 3)

    out_spec = pl.BlockSpec((1, 1), lambda i: (0, 0))

    outs = pl.pallas_call(
        _loss_kernel,
        grid=(G,),
        in_specs=[
            pl.BlockSpec((1, BN), lambda i: (0, i)),
            pl.BlockSpec((80, BN), lambda i: (0, i)),
            pl.BlockSpec((80, BN), lambda i: (0, i)),
            pl.BlockSpec((4, BN), lambda i: (0, i)),
            pl.BlockSpec((4, BN), lambda i: (0, i)),
            pl.BlockSpec((1, 1, BN), lambda i: (i, 0, 0)),
            pl.BlockSpec((1, 3), lambda i: (0, 0)),
        ],
        out_specs=[out_spec, out_spec, out_spec, out_spec],
        out_shape=[jax.ShapeDtypeStruct((1, 1), jnp.float32)] * 4,
        scratch_shapes=[
            pltpu.VMEM((4, CH), jnp.float32),
            pltpu.VMEM((8, CH), jnp.float32),
        ],
        compiler_params=pltpu.CompilerParams(
            dimension_semantics=("arbitrary",),
        ),
    )(conf_t, clsp_t, clst_t, boxp_t, boxt_t, fg3, aw2)

    oc, ocl, ob, ot = outs
    return (oc.reshape(()), ocl.reshape(()), ob.reshape(()), ot.reshape(()))
